# baseline (speedup 1.0000x reference)
import functools

import jax
import jax.numpy as jnp
from jax import lax
from jax.experimental import pallas as pl
from jax.experimental.pallas import tpu as pltpu
from jax.experimental.pallas import tpu_sc as plsc

VOCAB = 30000
EMB = 128
B = 4
S = 2048
EPS = 1e-12

NC = 2
NS = 16
NW = NC * NS
BH = B // 2
TOKH = BH * S
TPW = TOKH // NW
IDXW = 128


@functools.partial(
    pl.kernel,
    out_type=jax.ShapeDtypeStruct((TOKH, EMB), jnp.float32),
    mesh=plsc.VectorSubcoreMesh(core_axis_name="c", subcore_axis_name="s"),
    scratch_types=[
        pltpu.VMEM((1, IDXW), jnp.int32),
        pltpu.VMEM((TPW, EMB), jnp.float32),
        pltpu.SemaphoreType.DMA,
        pltpu.SemaphoreType.DMA,
    ],
)
def _gather_half(ids_hbm, w_hbm, out_hbm, idx_v, rows_v, gsem, wsem):
    cid = lax.axis_index("c")
    sid = lax.axis_index("s")
    wid = sid * NC + cid
    base = wid * TPW

    brow = lax.div(base, S)
    bcol = lax.rem(base, S)
    pltpu.sync_copy(ids_hbm.at[brow, pl.ds(bcol, IDXW)], idx_v.at[0])

    pltpu.async_copy(w_hbm.at[idx_v.at[0]], rows_v, gsem).wait()
    pltpu.async_copy(rows_v, out_hbm.at[pl.ds(base, TPW)], wsem).wait()


def _ln_body(mid_ref, pos_ref, tte_ref, g_ref, b_ref, o_ref):
    x = mid_ref[...] + pos_ref[...] + tte_ref[0:1, :]
    m = jnp.mean(x, axis=-1, keepdims=True)
    d = x - m
    var = jnp.mean(d * d, axis=-1, keepdims=True)
    o_ref[...] = (d * lax.rsqrt(var + EPS) * g_ref[...] + b_ref[...])[None]


def _ln_body_alias(mid_ref, pos_ref, tte_ref, g_ref, b_ref, prev_ref, o_ref):
    del prev_ref
    _ln_body(mid_ref, pos_ref, tte_ref, g_ref, b_ref, o_ref)


_ln_specs = [
    pl.BlockSpec((S, EMB), lambda i: (i, 0)),
    pl.BlockSpec((S, EMB), lambda i: (0, 0)),
    pl.BlockSpec((2, EMB), lambda i: (0, 0)),
    pl.BlockSpec((1, EMB), lambda i: (0, 0)),
    pl.BlockSpec((1, EMB), lambda i: (0, 0)),
]

_ln_h0 = pl.pallas_call(
    _ln_body,
    out_shape=jax.ShapeDtypeStruct((B, S, EMB), jnp.float32),
    grid=(BH,),
    in_specs=_ln_specs,
    out_specs=pl.BlockSpec((1, S, EMB), lambda i: (i, 0, 0)),
)

_ln_h1 = pl.pallas_call(
    _ln_body_alias,
    out_shape=jax.ShapeDtypeStruct((B, S, EMB), jnp.float32),
    grid=(BH,),
    in_specs=_ln_specs + [pl.BlockSpec(memory_space=pl.ANY)],
    out_specs=pl.BlockSpec((1, S, EMB), lambda i: (i + BH, 0, 0)),
    input_output_aliases={5: 0},
)


def kernel(input_ids, weight, token_type_embeddings, position_embeddings,
           ln_gamma, ln_beta):
    ids = input_ids.astype(jnp.int32)
    mid0 = _gather_half(ids[:BH], weight)
    mid1 = _gather_half(ids[BH:], weight)
    gamma2 = ln_gamma.reshape(1, EMB)
    beta2 = ln_beta.reshape(1, EMB)
    out0 = _ln_h0(mid0, position_embeddings, token_type_embeddings,
                  gamma2, beta2)
    return _ln_h1(mid1, position_embeddings, token_type_embeddings,
                  gamma2, beta2, out0)


# speedup vs baseline: 1.0159x; 1.0159x over previous
import functools

import jax
import jax.numpy as jnp
from jax import lax
from jax.experimental import pallas as pl
from jax.experimental.pallas import tpu as pltpu
from jax.experimental.pallas import tpu_sc as plsc

VOCAB = 30000
EMB = 128
B = 4
S = 2048
EPS = 1e-12

NC = 2
NS = 16
NW = NC * NS
TOK = B * S
TPW = TOK // NW
IDXW = 128
NIDX = TPW // IDXW


@functools.partial(
    pl.kernel,
    out_type=jax.ShapeDtypeStruct((TOK, EMB), jnp.float32),
    mesh=plsc.VectorSubcoreMesh(core_axis_name="c", subcore_axis_name="s"),
    scratch_types=[
        pltpu.VMEM((NIDX, IDXW), jnp.int32),
        pltpu.VMEM((TPW, EMB), jnp.float32),
        pltpu.SemaphoreType.DMA,
        pltpu.SemaphoreType.DMA,
        pltpu.SemaphoreType.DMA,
    ],
)
def _gather(ids_hbm, w_hbm, out_hbm, idx_v, rows_v, gsem0, gsem1, wsem):
    cid = lax.axis_index("c")
    sid = lax.axis_index("s")
    wid = sid * NC + cid
    base = wid * TPW

    brow = lax.div(base, S)
    bcol = lax.rem(base, S)
    for j in range(NIDX):
        pltpu.sync_copy(ids_hbm.at[brow, pl.ds(bcol + j * IDXW, IDXW)],
                        idx_v.at[j])

    gsems = [gsem0, gsem1]
    gcps = [
        pltpu.async_copy(w_hbm.at[idx_v.at[j]],
                         rows_v.at[pl.ds(j * IDXW, IDXW)], gsems[j])
        for j in range(NIDX)
    ]
    wcps = []
    for j in range(NIDX):
        gcps[j].wait()
        wcps.append(pltpu.async_copy(
            rows_v.at[pl.ds(j * IDXW, IDXW)],
            out_hbm.at[pl.ds(base + j * IDXW, IDXW)], wsem))
    for cp in wcps:
        cp.wait()


def _ln_body(mid_ref, pos_ref, tte_ref, g_ref, b_ref, o_ref):
    x = mid_ref[...] + pos_ref[...] + tte_ref[0:1, :]
    m = jnp.mean(x, axis=-1, keepdims=True)
    d = x - m
    var = jnp.mean(d * d, axis=-1, keepdims=True)
    o_ref[...] = (d * lax.rsqrt(var + EPS) * g_ref[...] + b_ref[...])[None]


SBLK = 1024
NSB = S // SBLK

_ln_call = pl.pallas_call(
    _ln_body,
    out_shape=jax.ShapeDtypeStruct((B, S, EMB), jnp.float32),
    grid=(NSB, B),
    in_specs=[
        pl.BlockSpec((SBLK, EMB), lambda j, b: (b * NSB + j, 0)),
        pl.BlockSpec((SBLK, EMB), lambda j, b: (j, 0)),
        pl.BlockSpec((2, EMB), lambda j, b: (0, 0)),
        pl.BlockSpec((1, EMB), lambda j, b: (0, 0)),
        pl.BlockSpec((1, EMB), lambda j, b: (0, 0)),
    ],
    out_specs=pl.BlockSpec((1, SBLK, EMB), lambda j, b: (b, j, 0)),
)


def kernel(input_ids, weight, token_type_embeddings, position_embeddings,
           ln_gamma, ln_beta):
    mid = _gather(input_ids.astype(jnp.int32), weight)
    return _ln_call(mid,
                    position_embeddings,
                    token_type_embeddings,
                    ln_gamma.reshape(1, EMB),
                    ln_beta.reshape(1, EMB))


# speedup vs baseline: 1.0964x; 1.0793x over previous
import functools

import jax
import jax.numpy as jnp
from jax import lax
from jax.experimental import pallas as pl
from jax.experimental.pallas import tpu as pltpu
from jax.experimental.pallas import tpu_sc as plsc

VOCAB = 30000
EMB = 128
B = 4
S = 2048
EPS = 1e-12

NC = 2
NS = 16
NW = NC * NS
TOK = B * S
TPW = TOK // NW
IDXW = 128
NIDX = TPW // IDXW


@functools.partial(
    pl.kernel,
    out_type=jax.ShapeDtypeStruct((TOK, EMB), jnp.float32),
    mesh=plsc.VectorSubcoreMesh(core_axis_name="c", subcore_axis_name="s"),
    scratch_types=[
        pltpu.VMEM((NIDX, IDXW), jnp.int32),
        pltpu.VMEM((TPW, EMB), jnp.float32),
        pltpu.SemaphoreType.DMA,
        pltpu.SemaphoreType.DMA,
        pltpu.SemaphoreType.DMA,
    ],
)
def _gather(ids_hbm, w_hbm, out_hbm, idx_v, rows_v, gsem0, gsem1, wsem):
    cid = lax.axis_index("c")
    sid = lax.axis_index("s")
    wid = sid * NC + cid
    base = wid * TPW

    pltpu.sync_copy(ids_hbm.at[pl.ds(wid * NIDX, NIDX)], idx_v)

    gsems = [gsem0, gsem1]
    gcps = [
        pltpu.async_copy(w_hbm.at[idx_v.at[j]],
                         rows_v.at[pl.ds(j * IDXW, IDXW)], gsems[j])
        for j in range(NIDX)
    ]
    wcps = []
    for j in range(NIDX):
        gcps[j].wait()
        wcps.append(pltpu.async_copy(
            rows_v.at[pl.ds(j * IDXW, IDXW)],
            out_hbm.at[pl.ds(base + j * IDXW, IDXW)], wsem))
    for cp in wcps:
        cp.wait()


def _ln_body(mid_ref, pos_ref, tte_ref, g_ref, b_ref, o_ref):
    x = mid_ref[...] + pos_ref[...] + tte_ref[0:1, :]
    m = jnp.mean(x, axis=-1, keepdims=True)
    d = x - m
    var = jnp.mean(d * d, axis=-1, keepdims=True)
    o_ref[...] = (d * lax.rsqrt(var + EPS) * g_ref[...] + b_ref[...])[None]


_ln_call = pl.pallas_call(
    _ln_body,
    out_shape=jax.ShapeDtypeStruct((B, S, EMB), jnp.float32),
    grid=(B,),
    in_specs=[
        pl.BlockSpec((S, EMB), lambda i: (i, 0)),
        pl.BlockSpec((S, EMB), lambda i: (0, 0)),
        pl.BlockSpec((2, EMB), lambda i: (0, 0)),
        pl.BlockSpec((1, EMB), lambda i: (0, 0)),
        pl.BlockSpec((1, EMB), lambda i: (0, 0)),
    ],
    out_specs=pl.BlockSpec((1, S, EMB), lambda i: (i, 0, 0)),
)


def kernel(input_ids, weight, token_type_embeddings, position_embeddings,
           ln_gamma, ln_beta):
    mid = _gather(input_ids.astype(jnp.int32).reshape(TOK // IDXW, IDXW),
                  weight)
    return _ln_call(mid,
                    position_embeddings,
                    token_type_embeddings,
                    ln_gamma.reshape(1, EMB),
                    ln_beta.reshape(1, EMB))
